# baseline (speedup 1.0000x reference)
import functools

import jax
import jax.numpy as jnp
from jax import lax
from jax.experimental import pallas as pl
from jax.experimental.pallas import tpu as pltpu
from jax.experimental.pallas import tpu_sc as plsc

_E = 16
_K = 2
_N = 8192
_D = 1024
_CAP = 2048
_NW = 32
_CHUNK = _N // _NW
_BLK = 32
_G = 16
_ZB = 32

_mesh = plsc.VectorSubcoreMesh(core_axis_name="c", subcore_axis_name="s")

def _wid():
    return lax.axis_index("s") * 2 + lax.axis_index("c")


def _lane_gather(x, idx):
    dnums = lax.GatherDimensionNumbers(
        offset_dims=(), collapsed_slice_dims=(0,), start_index_map=(0,))
    return lax.gather(x, idx[:, None], dnums, slice_sizes=(1,),
                      mode=lax.GatherScatterMode.PROMISE_IN_BOUNDS)


def _vmax_splat(x):
    iota = lax.iota(jnp.int32, 16)
    for sh in (1, 2, 4, 8):
        x = jnp.maximum(x, _lane_gather(x, iota ^ sh))
    return x


def _vmin_splat(x):
    iota = lax.iota(jnp.int32, 16)
    for sh in (1, 2, 4, 8):
        x = jnp.minimum(x, _lane_gather(x, iota ^ sh))
    return x


def _ffs_splat(mask):
    iota = lax.iota(jnp.int32, 16)
    return _vmin_splat(jnp.where(mask, iota, 16))


def _gate_body(score_hbm, hot_hbm, cc_hbm, score_v, hot_v, cnt_v):
    wid = _wid()
    base = wid * _CHUNK
    pltpu.sync_copy(score_hbm.at[pl.ds(base, _CHUNK)], score_v)
    iota = lax.iota(jnp.int32, 16)
    neg_inf = jnp.float32(float("-inf"))

    def top2(s):
        i1 = _ffs_splat(s == _vmax_splat(s))
        m1 = iota == i1
        s2 = jnp.where(m1, neg_inf, s)
        i2 = _ffs_splat(s2 == _vmax_splat(s2))
        return jnp.where(m1 | (iota == i2), 1, 0).astype(jnp.int32)

    def tok(i, cnt):
        t = 2 * i
        hot_a = top2(score_v[t])
        hot_b = top2(score_v[t + 1])
        hot_v[pl.ds(t * _E, _E)] = hot_a
        hot_v[pl.ds((t + 1) * _E, _E)] = hot_b
        return cnt + hot_a + hot_b

    cnt = lax.fori_loop(0, _CHUNK // 2, tok, jnp.zeros((16,), jnp.int32))
    cnt_v[...] = cnt
    pltpu.sync_copy(hot_v,
                    hot_hbm.at[pl.ds(pl.multiple_of(base * _E, 4096),
                                     _CHUNK * _E)])
    pltpu.sync_copy(cnt_v, cc_hbm.at[pl.ds(pl.multiple_of(wid * _E, 16), _E)])


_gate = functools.partial(
    pl.kernel,
    out_type=(
        jax.ShapeDtypeStruct((_N * _E,), jnp.int32),
        jax.ShapeDtypeStruct((_NW * _E,), jnp.int32),
    ),
    mesh=_mesh,
    scratch_types=[
        pltpu.VMEM((_CHUNK, _E), jnp.float32),
        pltpu.VMEM((_CHUNK * _E,), jnp.int32),
        pltpu.VMEM((_E,), jnp.int32),
    ],
)(_gate_body)


def _scatter_body(in_hbm, hot_hbm, cc_hbm, out_hbm, counts_hbm,
                  cc_v, hot_v, buf0, buf1, zbuf, c16_v, c32_v, idx_v,
                  zidx_v, sem_l, sem_s, sem_z):
    wid = _wid()
    base = wid * _CHUNK
    ld0 = pltpu.async_copy(
        in_hbm.at[pl.ds(pl.multiple_of(base, 32), _BLK)], buf0, sem_l)
    cpc = pltpu.async_copy(cc_hbm, cc_v, sem_s)
    cph = pltpu.async_copy(
        hot_hbm.at[pl.ds(pl.multiple_of(base * _E, 4096), _CHUNK * _E)],
        hot_v, sem_s)
    cpc.wait()
    cph.wait()
    iota = lax.iota(jnp.int32, 16)
    zeros16 = jnp.zeros((16,), jnp.int32)

    def accum(w, carry):
        tot, myb = carry
        row = cc_v[pl.ds(w * _E, _E)]
        return tot + row, myb + row * (w < wid).astype(jnp.int32)

    totals, mybase = lax.fori_loop(0, _NW, accum, (zeros16, zeros16))
    c16_v[...] = totals
    c32_v[pl.ds(0, 16)] = totals
    c32_v[pl.ds(16, 16)] = totals

    @pl.when(wid == 0)
    def _():
        pltpu.sync_copy(c16_v, counts_hbm)

    zv = jnp.zeros((16,), jnp.float32)

    def zb(i, c):
        zbuf[i >> 6, pl.ds((i & 63) * 16, 16)] = zv
        return c

    lax.fori_loop(0, (_ZB * _D) // 16, zb, 0)

    e = wid & (_E - 1)
    par = (wid >= _E).astype(jnp.int32)
    ce = c32_v[pl.ds(e, 16)][0]
    c = jnp.minimum(ce, _CAP)
    g8 = jnp.minimum((8 - (c & 7)) & 7, _CAP - c)

    @pl.when((wid < _E) & (c < _CAP))
    def _():
        zidx_v[...] = e * _CAP + jnp.minimum(c + iota, _CAP - 1)
        pltpu.sync_copy(zbuf.at[pl.ds(0, 16)], out_hbm.at[zidx_v])

    a8 = c + g8
    g32 = jnp.minimum((32 - (a8 & 31)) & 31, _CAP - a8)

    @pl.when(wid < _E)
    def _():
        for sz in (16, 8):
            off = a8 + (g32 & ~(2 * sz - 1))

            @pl.when((g32 & sz) != 0)
            def _(off=off, sz=sz):
                pltpu.sync_copy(
                    zbuf.at[pl.ds(0, sz)],
                    out_hbm.at[pl.ds(pl.multiple_of(e * _CAP + off, 8), sz)])

    start = a8 + g32
    nblk = (_CAP - start) >> 5
    nmine = (nblk + 1 - par) >> 1

    def zfire(j, carry):
        row = start + (2 * j + par) * _ZB
        pltpu.async_copy(
            zbuf, out_hbm.at[pl.ds(pl.multiple_of(e * _CAP + row, 8), _ZB)],
            sem_z)
        return carry

    lax.fori_loop(0, nmine, zfire, 0)

    nb = _CHUNK // _BLK
    bufs = (buf0, buf1)
    lds = [None] * nb
    scs = [None] * nb
    lds[0] = ld0
    cnt = mybase
    for b in range(nb):
        lds[b].wait()
        if b >= 1:
            for cp in scs[b - 1]:
                cp.wait()
        if b + 1 < nb:
            lds[b + 1] = pltpu.async_copy(
                in_hbm.at[pl.ds(pl.multiple_of(base + (b + 1) * _BLK, 32),
                                _BLK)],
                bufs[(b + 1) & 1], sem_l)
        for g in range(_BLK // _G):
            one = jnp.full((16,), 1, jnp.int32)

            def onehots(h):
                i1 = _vmin_splat(iota + (one - h) * 64)
                eq1 = one - jnp.minimum(jnp.abs(iota - i1), one)
                h2 = h * (one - eq1)
                i2 = _vmin_splat(iota + (one - h2) * 64)
                eq2 = one - jnp.minimum(jnp.abs(iota - i2), one)
                return eq1, eq2

            def tok(i, carry):
                cnt, d1a, d2a = carry
                t = b * _BLK + g * _G + 2 * i
                ha = hot_v[pl.ds(t * _E, _E)]
                hb = hot_v[pl.ds((t + 1) * _E, _E)]
                eq1a, eq2a = onehots(ha)
                eq1b, eq2b = onehots(hb)
                posa = jnp.minimum(cnt, _CAP - 1)
                cntm = cnt + ha
                posb = jnp.minimum(cntm, _CAP - 1)
                dva = iota * _CAP + posa
                dvb = iota * _CAP + posb
                d1a_ = _vmax_splat(dva * eq1a)
                d2a_ = _vmax_splat(dva * eq2a)
                d1b_ = _vmax_splat(dvb * eq1b)
                d2b_ = _vmax_splat(dvb * eq2b)
                sa = one - jnp.minimum(jnp.abs(iota - 2 * i), one)
                sb = one - jnp.minimum(jnp.abs(iota - (2 * i + 1)), one)
                d1a = d1a + (d1a_ - d1a) * sa + (d1b_ - d1a) * sb
                d2a = d2a + (d2a_ - d2a) * sa + (d2b_ - d2a) * sb
                return (cntm + hb, d1a, d2a)

            cnt, d1a, d2a = lax.fori_loop(0, _G // 2, tok,
                                          (cnt, zeros16, zeros16))
            p = b & 1
            idx_v[2 * p, pl.ds(g * _G, _G)] = d1a
            idx_v[2 * p + 1, pl.ds(g * _G, _G)] = d2a
        p = b & 1
        scs[b] = [
            pltpu.async_copy(bufs[p], out_hbm.at[idx_v.at[2 * p]], sem_s),
            pltpu.async_copy(bufs[p], out_hbm.at[idx_v.at[2 * p + 1]], sem_s),
        ]
    for cp in scs[nb - 1]:
        cp.wait()

    def zdrain(j, carry):
        pltpu.make_async_copy(zbuf, out_hbm.at[pl.ds(0, _ZB)], sem_z).wait()
        return carry

    lax.fori_loop(0, nmine, zdrain, 0)


_scatter = functools.partial(
    pl.kernel,
    out_type=(
        jax.ShapeDtypeStruct((_E * _CAP, _D), jnp.float32),
        jax.ShapeDtypeStruct((_E,), jnp.int32),
    ),
    mesh=_mesh,
    scratch_types=[
        pltpu.VMEM((_NW * _E,), jnp.int32),
        pltpu.VMEM((_CHUNK * _E,), jnp.int32),
        pltpu.VMEM((_BLK, _D), jnp.float32),
        pltpu.VMEM((_BLK, _D), jnp.float32),
        pltpu.VMEM((_ZB, _D), jnp.float32),
        pltpu.VMEM((_E,), jnp.int32),
        pltpu.VMEM((2 * _E,), jnp.int32),
        pltpu.VMEM((4, _BLK), jnp.int32),
        pltpu.VMEM((16,), jnp.int32),
        pltpu.SemaphoreType.DMA,
        pltpu.SemaphoreType.DMA,
        pltpu.SemaphoreType.DMA,
    ],
)(_scatter_body)


@jax.jit
def kernel(in_flow, score):
    hot, cc = _gate(score)
    out, counts = _scatter(in_flow, hot, cc)
    return out.reshape(_E, _CAP, _D), counts


# speedup vs baseline: 1.0092x; 1.0092x over previous
import functools

import jax
import jax.numpy as jnp
from jax import lax
from jax.experimental import pallas as pl
from jax.experimental.pallas import tpu as pltpu
from jax.experimental.pallas import tpu_sc as plsc

_E = 16
_K = 2
_N = 8192
_D = 1024
_CAP = 2048
_NW = 32
_CHUNK = _N // _NW
_BLK = 32
_G = 16
_ZB = 32

_mesh = plsc.VectorSubcoreMesh(core_axis_name="c", subcore_axis_name="s")

def _wid():
    return lax.axis_index("s") * 2 + lax.axis_index("c")


def _lane_gather(x, idx):
    dnums = lax.GatherDimensionNumbers(
        offset_dims=(), collapsed_slice_dims=(0,), start_index_map=(0,))
    return lax.gather(x, idx[:, None], dnums, slice_sizes=(1,),
                      mode=lax.GatherScatterMode.PROMISE_IN_BOUNDS)


def _vmax_splat(x):
    iota = lax.iota(jnp.int32, 16)
    for sh in (1, 2, 4, 8):
        x = jnp.maximum(x, _lane_gather(x, iota ^ sh))
    return x


def _vmin_splat(x):
    iota = lax.iota(jnp.int32, 16)
    for sh in (1, 2, 4, 8):
        x = jnp.minimum(x, _lane_gather(x, iota ^ sh))
    return x


def _ffs_splat(mask):
    iota = lax.iota(jnp.int32, 16)
    return _vmin_splat(jnp.where(mask, iota, 16))


def _gate_body(score_hbm, hot_hbm, cc_hbm, score_v, hot_v, cnt_v):
    wid = _wid()
    base = wid * _CHUNK
    pltpu.sync_copy(score_hbm.at[pl.ds(base, _CHUNK)], score_v)
    iota = lax.iota(jnp.int32, 16)
    neg_inf = jnp.float32(float("-inf"))

    def top2(s):
        i1 = _ffs_splat(s == _vmax_splat(s))
        m1 = iota == i1
        s2 = jnp.where(m1, neg_inf, s)
        i2 = _ffs_splat(s2 == _vmax_splat(s2))
        return jnp.where(m1 | (iota == i2), 1, 0).astype(jnp.int32)

    def tok(i, cnt):
        t = 2 * i
        hot_a = top2(score_v[t])
        hot_b = top2(score_v[t + 1])
        hot_v[pl.ds(t * _E, _E)] = hot_a
        hot_v[pl.ds((t + 1) * _E, _E)] = hot_b
        return cnt + hot_a + hot_b

    cnt = lax.fori_loop(0, _CHUNK // 2, tok, jnp.zeros((16,), jnp.int32))
    cnt_v[...] = cnt
    pltpu.sync_copy(hot_v,
                    hot_hbm.at[pl.ds(pl.multiple_of(base * _E, 4096),
                                     _CHUNK * _E)])
    pltpu.sync_copy(cnt_v, cc_hbm.at[pl.ds(pl.multiple_of(wid * _E, 16), _E)])


_gate = functools.partial(
    pl.kernel,
    out_type=(
        jax.ShapeDtypeStruct((_N * _E,), jnp.int32),
        jax.ShapeDtypeStruct((_NW * _E,), jnp.int32),
    ),
    mesh=_mesh,
    scratch_types=[
        pltpu.VMEM((_CHUNK, _E), jnp.float32),
        pltpu.VMEM((_CHUNK * _E,), jnp.int32),
        pltpu.VMEM((_E,), jnp.int32),
    ],
)(_gate_body)


def _scatter_body(in_hbm, hot_hbm, cc_hbm, out_hbm, counts_hbm,
                  cc_v, hot_v, buf0, buf1, zbuf, c16_v, c32_v, idx_v,
                  zidx_v, sem_l, sem_s, sem_z):
    wid = _wid()
    base = wid * _CHUNK
    ld0 = pltpu.async_copy(
        in_hbm.at[pl.ds(pl.multiple_of(base, 32), _BLK)], buf0, sem_l)
    cpc = pltpu.async_copy(cc_hbm, cc_v, sem_s)
    cph = pltpu.async_copy(
        hot_hbm.at[pl.ds(pl.multiple_of(base * _E, 4096), _CHUNK * _E)],
        hot_v, sem_s)
    cpc.wait()
    cph.wait()
    iota = lax.iota(jnp.int32, 16)
    zeros16 = jnp.zeros((16,), jnp.int32)

    def accum(w, carry):
        tot, myb = carry
        row = cc_v[pl.ds(w * _E, _E)]
        return tot + row, myb + row * (w < wid).astype(jnp.int32)

    totals, mybase = lax.fori_loop(0, _NW, accum, (zeros16, zeros16))
    c16_v[...] = totals
    c32_v[pl.ds(0, 16)] = totals
    c32_v[pl.ds(16, 16)] = totals

    @pl.when(wid == 0)
    def _():
        pltpu.sync_copy(c16_v, counts_hbm)

    zv = jnp.zeros((16,), jnp.float32)

    def zb(i, c):
        zbuf[i >> 6, pl.ds((i & 63) * 16, 16)] = zv
        return c

    lax.fori_loop(0, (_ZB * _D) // 16, zb, 0)

    e = wid & (_E - 1)
    par = (wid >= _E).astype(jnp.int32)
    ce = c32_v[pl.ds(e, 16)][0]
    c = jnp.minimum(ce, _CAP)
    g8 = jnp.minimum((8 - (c & 7)) & 7, _CAP - c)

    @pl.when((wid < _E) & (c < _CAP))
    def _():
        zidx_v[...] = e * _CAP + jnp.minimum(c + iota, _CAP - 1)
        pltpu.sync_copy(zbuf.at[pl.ds(0, 16)], out_hbm.at[zidx_v])

    a8 = c + g8
    g32 = jnp.minimum((32 - (a8 & 31)) & 31, _CAP - a8)

    @pl.when(wid < _E)
    def _():
        for sz in (16, 8):
            off = a8 + (g32 & ~(2 * sz - 1))

            @pl.when((g32 & sz) != 0)
            def _(off=off, sz=sz):
                pltpu.sync_copy(
                    zbuf.at[pl.ds(0, sz)],
                    out_hbm.at[pl.ds(pl.multiple_of(e * _CAP + off, 8), sz)])

    start = a8 + g32
    nblk = (_CAP - start) >> 5
    nmine = (nblk + 1 - par) >> 1

    def zfire(j, carry):
        row = start + (2 * j + par) * _ZB
        pltpu.async_copy(
            zbuf, out_hbm.at[pl.ds(pl.multiple_of(e * _CAP + row, 8), _ZB)],
            sem_z)
        return carry

    lax.fori_loop(0, nmine, zfire, 0)

    nb = _CHUNK // _BLK
    bufs = (buf0, buf1)
    lds = [None] * nb
    scs = [None] * nb
    lds[0] = ld0
    cnt = mybase
    for b in range(nb):
        lds[b].wait()
        if b >= 1:
            for cp in scs[b - 1]:
                cp.wait()
        if b + 1 < nb:
            lds[b + 1] = pltpu.async_copy(
                in_hbm.at[pl.ds(pl.multiple_of(base + (b + 1) * _BLK, 32),
                                _BLK)],
                bufs[(b + 1) & 1], sem_l)
        for g in range(_BLK // _G):
            def tok(t16, carry):
                cnt, d1a, d2a = carry
                t = b * _BLK + g * _G + t16
                h = hot_v[pl.ds(t * _E, _E)]
                pos = jnp.minimum(cnt, _CAP - 1)
                one = jnp.full((16,), 1, jnp.int32)
                i1 = _vmin_splat(iota + (one - h) * 64)
                eq1 = one - jnp.minimum(jnp.abs(iota - i1), one)
                h2 = h * (one - eq1)
                i2 = _vmin_splat(iota + (one - h2) * 64)
                eq2 = one - jnp.minimum(jnp.abs(iota - i2), one)
                dvec = iota * _CAP + pos
                d1 = _vmax_splat(dvec * eq1)
                d2 = _vmax_splat(dvec * eq2)
                selv = one - jnp.minimum(jnp.abs(iota - t16), one)
                return (cnt + h,
                        d1a + (d1 - d1a) * selv,
                        d2a + (d2 - d2a) * selv)

            cnt, d1a, d2a = lax.fori_loop(0, _G, tok, (cnt, zeros16, zeros16))
            p = b & 1
            idx_v[2 * p, pl.ds(g * _G, _G)] = d1a
            idx_v[2 * p + 1, pl.ds(g * _G, _G)] = d2a
        p = b & 1
        scs[b] = [
            pltpu.async_copy(bufs[p], out_hbm.at[idx_v.at[2 * p]], sem_s),
            pltpu.async_copy(bufs[p], out_hbm.at[idx_v.at[2 * p + 1]], sem_s),
        ]
    for cp in scs[nb - 1]:
        cp.wait()

    def zdrain(j, carry):
        pltpu.make_async_copy(zbuf, out_hbm.at[pl.ds(0, _ZB)], sem_z).wait()
        return carry

    lax.fori_loop(0, nmine, zdrain, 0)


_scatter = functools.partial(
    pl.kernel,
    out_type=(
        jax.ShapeDtypeStruct((_E * _CAP, _D), jnp.float32),
        jax.ShapeDtypeStruct((_E,), jnp.int32),
    ),
    mesh=_mesh,
    scratch_types=[
        pltpu.VMEM((_NW * _E,), jnp.int32),
        pltpu.VMEM((_CHUNK * _E,), jnp.int32),
        pltpu.VMEM((_BLK, _D), jnp.float32),
        pltpu.VMEM((_BLK, _D), jnp.float32),
        pltpu.VMEM((_ZB, _D), jnp.float32),
        pltpu.VMEM((_E,), jnp.int32),
        pltpu.VMEM((2 * _E,), jnp.int32),
        pltpu.VMEM((4, _BLK), jnp.int32),
        pltpu.VMEM((16,), jnp.int32),
        pltpu.SemaphoreType.DMA,
        pltpu.SemaphoreType.DMA,
        pltpu.SemaphoreType.DMA,
    ],
)(_scatter_body)


@jax.jit
def kernel(in_flow, score):
    hot, cc = _gate(score)
    out, counts = _scatter(in_flow, hot, cc)
    return out.reshape(_E, _CAP, _D), counts
